# baseline (speedup 1.0000x reference)
import functools

import jax
import jax.numpy as jnp
from jax import lax
from jax.experimental import pallas as pl
from jax.experimental.pallas import tpu as pltpu
from jax.experimental.pallas import tpu_sc as plsc

D = 128
NUM_CORES = 2
NUM_SUBCORES = 16
NW = NUM_CORES * NUM_SUBCORES


@jax.jit
def _lookup_t(idx_t, embeddings):
    row, nb = idx_t.shape
    per_tile = nb // NW
    assert nb % NW == 0 and row % 2 == 0
    mesh = plsc.VectorSubcoreMesh(core_axis_name="c", subcore_axis_name="s")

    @functools.partial(
        pl.kernel,
        mesh=mesh,
        out_type=jax.ShapeDtypeStruct((row, nb, D), jnp.float32),
        scratch_types=[
            pltpu.VMEM((row, per_tile), jnp.int32),
            pltpu.VMEM((per_tile, D), jnp.float32),
            pltpu.VMEM((per_tile, D), jnp.float32),
            pltpu.VMEM((per_tile, D), jnp.float32),
            pltpu.VMEM((per_tile, D), jnp.float32),
            pltpu.VMEM((per_tile, D), jnp.float32),
            pltpu.VMEM((per_tile, D), jnp.float32),
            pltpu.SemaphoreType.DMA,
            pltpu.SemaphoreType.DMA,
            pltpu.SemaphoreType.DMA,
            pltpu.SemaphoreType.DMA,
            pltpu.SemaphoreType.DMA,
            pltpu.SemaphoreType.DMA,
            pltpu.SemaphoreType.DMA,
            pltpu.SemaphoreType.DMA,
            pltpu.SemaphoreType.DMA,
            pltpu.SemaphoreType.DMA,
            pltpu.SemaphoreType.DMA,
            pltpu.SemaphoreType.DMA,
        ],
    )
    def k(idx_hbm, table_hbm, out_hbm, idx_v,
          b_0, b_1, b_2, b_3, b_4, b_5,
          s_0, s_1, s_2, s_3, s_4, s_5,
          o_0, o_1, o_2, o_3, o_4, o_5):
        wid = lax.axis_index("s") * NUM_CORES + lax.axis_index("c")
        b0 = wid * per_tile
        pltpu.sync_copy(idx_hbm.at[:, pl.ds(b0, per_tile)], idx_v)

        bufs = (b_0, b_1, b_2, b_3, b_4, b_5)
        gs = (s_0, s_1, s_2, s_3, s_4, s_5)
        os_ = (o_0, o_1, o_2, o_3, o_4, o_5)
        nbuf = 6

        def gather(j, p):
            return pltpu.make_async_copy(table_hbm.at[idx_v.at[j]], bufs[p], gs[p])

        def ocopy(j, p):
            return pltpu.make_async_copy(
                bufs[p], out_hbm.at[j, pl.ds(b0, per_tile)], os_[p]
            )

        for j in range(nbuf - 1):
            gather(j, j).start()
        gather(0, 0).wait()
        ocopy(0, 0).start()
        gather(nbuf - 1, nbuf - 1).start()

        def body(h, carry):
            for p6 in range(nbuf):
                j = nbuf * h + p6 + 1
                p = (p6 + 1) % nbuf
                q = (p + nbuf - 1) % nbuf
                gather(j, p).wait()
                ocopy(j, p).start()

                @pl.when(j + nbuf - 1 < row)
                def _():
                    ocopy(j - 1, q).wait()
                    gather(j + nbuf - 1, q).start()
            return carry

        lax.fori_loop(0, (row - 2) // nbuf, body, 0)
        j = row - 1
        gather(j, j % nbuf).wait()
        ocopy(j, j % nbuf).start()
        for c in range(row - nbuf, row):
            ocopy(c, c % nbuf).wait()

    return k(idx_t, embeddings)


def kernel(inputs, embeddings):
    idx_t = jnp.transpose(inputs.astype(jnp.int32))
    out_t = _lookup_t(idx_t, embeddings)
    return jnp.transpose(out_t, (1, 0, 2))


# speedup vs baseline: 1.0012x; 1.0012x over previous
import functools

import jax
import jax.numpy as jnp
from jax import lax
from jax.experimental import pallas as pl
from jax.experimental.pallas import tpu as pltpu
from jax.experimental.pallas import tpu_sc as plsc

D = 128
NUM_CORES = 2
NUM_SUBCORES = 16
NW = NUM_CORES * NUM_SUBCORES
NBUF = 6


@jax.jit
def _lookup_t(idx_t, embeddings):
    row, nb = idx_t.shape
    per_tile = nb // NW
    assert nb % NW == 0 and (row - 2) % NBUF == 0
    mesh = plsc.VectorSubcoreMesh(core_axis_name="c", subcore_axis_name="s")

    @functools.partial(
        pl.kernel,
        mesh=mesh,
        out_type=jax.ShapeDtypeStruct((row, nb, D), jnp.float32),
        scratch_types=[
            pltpu.VMEM((row, per_tile), jnp.int32),
            pltpu.VMEM((per_tile, D), jnp.float32),
            pltpu.VMEM((per_tile, D), jnp.float32),
            pltpu.VMEM((per_tile, D), jnp.float32),
            pltpu.VMEM((per_tile, D), jnp.float32),
            pltpu.VMEM((per_tile, D), jnp.float32),
            pltpu.VMEM((per_tile, D), jnp.float32),
            pltpu.SemaphoreType.DMA,
            pltpu.SemaphoreType.DMA,
            pltpu.SemaphoreType.DMA,
            pltpu.SemaphoreType.DMA,
            pltpu.SemaphoreType.DMA,
            pltpu.SemaphoreType.DMA,
            pltpu.SemaphoreType.DMA,
            pltpu.SemaphoreType.DMA,
            pltpu.SemaphoreType.DMA,
            pltpu.SemaphoreType.DMA,
            pltpu.SemaphoreType.DMA,
            pltpu.SemaphoreType.DMA,
        ],
    )
    def k(idx_hbm, table_hbm, out_hbm, idx_v,
          b_0, b_1, b_2, b_3, b_4, b_5,
          s_0, s_1, s_2, s_3, s_4, s_5,
          o_0, o_1, o_2, o_3, o_4, o_5):
        wid = lax.axis_index("s") * NUM_CORES + lax.axis_index("c")
        b0 = wid * per_tile
        pltpu.sync_copy(idx_hbm.at[:, pl.ds(b0, per_tile)], idx_v)

        bufs = (b_0, b_1, b_2, b_3, b_4, b_5)
        gs = (s_0, s_1, s_2, s_3, s_4, s_5)
        os_ = (o_0, o_1, o_2, o_3, o_4, o_5)
        nbuf = NBUF

        def gather(j, p):
            return pltpu.make_async_copy(table_hbm.at[idx_v.at[j]], bufs[p], gs[p])

        def ocopy(j, p):
            return pltpu.make_async_copy(
                bufs[p], out_hbm.at[j, pl.ds(b0, per_tile)], os_[p]
            )

        for j in range(nbuf - 1):
            gather(j, j).start()
        gather(0, 0).wait()
        ocopy(0, 0).start()
        gather(nbuf - 1, nbuf - 1).start()

        def body(h, carry):
            for p6 in range(nbuf):
                j = nbuf * h + p6 + 1
                p = (p6 + 1) % nbuf
                q = (p + nbuf - 1) % nbuf
                gather(j, p).wait()
                ocopy(j, p).start()

                @pl.when(j + nbuf - 1 < row)
                def _():
                    ocopy(j - 1, q).wait()
                    gather(j + nbuf - 1, q).start()
            return carry

        lax.fori_loop(0, (row - 2) // nbuf, body, 0)
        j = row - 1
        gather(j, j % nbuf).wait()
        ocopy(j, j % nbuf).start()
        for c in range(row - nbuf, row):
            ocopy(c, c % nbuf).wait()

    return k(idx_t, embeddings)


def kernel(inputs, embeddings):
    idx_t = jnp.transpose(inputs.astype(jnp.int32))
    out_t = _lookup_t(idx_t, embeddings)
    return jnp.transpose(out_t, (1, 0, 2))
